# Initial kernel scaffold; baseline (speedup 1.0000x reference)
import jax
import jax.numpy as jnp
from jax.experimental import pallas as pl


def kernel(features_1, edge_index_1, features_2, edge_index_2, W1, b1, W2, b2, W3, b3):
    raise NotImplementedError("write your pallas kernel here")


# speedup vs baseline: 16.6400x; 16.6400x over previous
import functools

import jax
import jax.numpy as jnp
from jax import lax
from jax.experimental import pallas as pl
from jax.experimental.pallas import tpu as pltpu
from jax.experimental.pallas import tpu_sc as plsc

N = 10000
E = 320000
D = 128
F1, F2, F3 = 256, 128, 64

NC = 2
NS = 16
EPC = E // NS
CK = 80
NCH = EPC // CK
RPT0 = 624
RPT_LAST = 640
ROW_LAST = (NS - 1) * RPT0
DEGW = 8


def _tile_slab_copy(s, make_src, make_dst):
    @pl.when(s < NS - 1)
    def _():
        row0 = s * RPT0
        pltpu.sync_copy(make_src(row0, RPT0), make_dst(row0, RPT0))

    @pl.when(s == NS - 1)
    def _():
        pltpu.sync_copy(make_src(ROW_LAST, RPT_LAST),
                        make_dst(ROW_LAST, RPT_LAST))

_sc_mesh = dict(core_axis_name="c", subcore_axis_name="s")


@functools.partial(
    pl.kernel,
    mesh=plsc.VectorSubcoreMesh(**_sc_mesh),
    out_type=jax.ShapeDtypeStruct((2 * N, DEGW), jnp.float32),
    scratch_types=[
        pltpu.VMEM((CK,), jnp.int32),
        pltpu.VMEM((CK, DEGW), jnp.float32),
        pltpu.VMEM_SHARED((N, DEGW), jnp.float32),
        pltpu.SemaphoreType.DMA,
    ],
)
def _deg_kernel(dsts_hbm, ones_hbm, deg_hbm, idx_v, ones_v, acc_sh, sem):
    c = lax.axis_index("c")
    s = lax.axis_index("s")
    _tile_slab_copy(s,
                    lambda r0, nr: ones_hbm.at[pl.ds(0, nr)],
                    lambda r0, nr: acc_sh.at[pl.ds(r0, nr)])
    pltpu.sync_copy(ones_hbm.at[pl.ds(0, CK)], ones_v)
    plsc.subcore_barrier()

    def body(k, carry):
        off = c * E + s * EPC + k * CK
        pltpu.sync_copy(dsts_hbm.at[pl.ds(off, CK)], idx_v)
        pltpu.sync_copy(ones_v, acc_sh.at[idx_v], add=True)
        return carry

    lax.fori_loop(0, NCH, body, 0)
    plsc.subcore_barrier()
    _tile_slab_copy(s,
                    lambda r0, nr: acc_sh.at[pl.ds(r0, nr)],
                    lambda r0, nr: deg_hbm.at[pl.ds(c * N + r0, nr)])


@functools.partial(
    pl.kernel,
    mesh=plsc.VectorSubcoreMesh(**_sc_mesh),
    out_type=jax.ShapeDtypeStruct((2 * N, D), jnp.float32),
    scratch_types=[
        pltpu.VMEM((CK,), jnp.int32),
        pltpu.VMEM((CK,), jnp.int32),
        pltpu.VMEM((CK, D), jnp.float32),
        pltpu.VMEM_SHARED((N, D), jnp.float32),
        pltpu.SemaphoreType.DMA,
    ],
)
def _agg_kernel(srcs_hbm, dsts_hbm, y_hbm, gsum_hbm, src_v, dst_v, rows_v,
                acc_sh, sem):
    c = lax.axis_index("c")
    s = lax.axis_index("s")
    _tile_slab_copy(s,
                    lambda r0, nr: y_hbm.at[pl.ds(c * N + r0, nr)],
                    lambda r0, nr: acc_sh.at[pl.ds(r0, nr)])
    plsc.subcore_barrier()

    def body(k, carry):
        off = c * E + s * EPC + k * CK
        pltpu.sync_copy(srcs_hbm.at[pl.ds(off, CK)], src_v)
        pltpu.sync_copy(dsts_hbm.at[pl.ds(off, CK)], dst_v)
        pltpu.async_copy(y_hbm.at[src_v], rows_v, sem).wait()
        pltpu.sync_copy(rows_v, acc_sh.at[dst_v], add=True)
        return carry

    lax.fori_loop(0, NCH, body, 0)
    plsc.subcore_barrier()
    _tile_slab_copy(s,
                    lambda r0, nr: acc_sh.at[pl.ds(r0, nr)],
                    lambda r0, nr: gsum_hbm.at[pl.ds(c * N + r0, nr)])


_PREP_B = 2 * N // 10


def _prep_body(x_ref, deg_ref, y_ref):
    y_ref[...] = x_ref[...] * lax.rsqrt(deg_ref[:, 0:1])


_prep = pl.pallas_call(
    _prep_body,
    grid=(10,),
    in_specs=[
        pl.BlockSpec((_PREP_B, D), lambda i: (i, 0)),
        pl.BlockSpec((_PREP_B, DEGW), lambda i: (i, 0)),
    ],
    out_specs=pl.BlockSpec((_PREP_B, D), lambda i: (i, 0)),
    out_shape=jax.ShapeDtypeStruct((2 * N, D), jnp.float32),
)


_BR = 1000
_NBPG = N // _BR


def _mlp_body(g_ref, deg_ref, w1_ref, b1_ref, w2_ref, b2_ref, w3_ref, b3_ref,
              out_ref):
    i = pl.program_id(0)
    g = i // _NBPG
    rows = g_ref[...] * lax.rsqrt(deg_ref[:, 0:1])
    h1 = jnp.maximum(
        lax.dot_general(rows, w1_ref[...], (((1,), (0,)), ((), ())),
                        preferred_element_type=jnp.float32) + b1_ref[...], 0.0)
    h2 = jnp.maximum(
        lax.dot_general(h1, w2_ref[...], (((1,), (1,)), ((), ())),
                        preferred_element_type=jnp.float32) + b2_ref[...], 0.0)
    h3 = jnp.maximum(
        lax.dot_general(h2, w3_ref[...], (((1,), (1,)), ((), ())),
                        preferred_element_type=jnp.float32) + b3_ref[...], 0.0)
    part = jnp.sum(h3, axis=0, keepdims=True)

    @pl.when(i == 0)
    def _():
        out_ref[...] = jnp.zeros_like(out_ref)

    rowmask = (lax.broadcasted_iota(jnp.int32, (2, 1), 0) == g)
    out_ref[...] += jnp.where(rowmask, part, 0.0)

    @pl.when(i == 2 * _NBPG - 1)
    def _():
        out_ref[...] *= (1.0 / N)


_mlp = pl.pallas_call(
    _mlp_body,
    grid=(2 * _NBPG,),
    in_specs=[
        pl.BlockSpec((_BR, D), lambda i: (i, 0)),
        pl.BlockSpec((_BR, DEGW), lambda i: (i, 0)),
        pl.BlockSpec((D, F1), lambda i: (0, 0)),
        pl.BlockSpec((1, F1), lambda i: (0, 0)),
        pl.BlockSpec((F2, F1), lambda i: (0, 0)),
        pl.BlockSpec((1, F2), lambda i: (0, 0)),
        pl.BlockSpec((F3, F2), lambda i: (0, 0)),
        pl.BlockSpec((1, F3), lambda i: (0, 0)),
    ],
    out_specs=pl.BlockSpec((2, F3), lambda i: (0, 0)),
    out_shape=jax.ShapeDtypeStruct((2, F3), jnp.float32),
)


def kernel(features_1, edge_index_1, features_2, edge_index_2,
           W1, b1, W2, b2, W3, b3):
    xs = jnp.concatenate([features_1, features_2], axis=0)
    srcs = jnp.concatenate([edge_index_1[0], edge_index_2[0] + N])
    dsts = jnp.concatenate([edge_index_1[1], edge_index_2[1]])
    ones = jnp.ones((N, DEGW), jnp.float32)

    deg = _deg_kernel(dsts, ones)
    y = _prep(xs, deg)
    gsum = _agg_kernel(srcs, dsts, y)
    out = _mlp(gsum, deg, W1, b1[None, :], W2, b2[None, :], W3, b3[None, :])
    return (out[0].reshape(F3, 1), out[1].reshape(F3, 1))
